# baseline (speedup 1.0000x reference)
import functools

import jax
import jax.numpy as jnp
from jax import lax
from jax.experimental import pallas as pl
from jax.experimental.pallas import tpu as pltpu
from jax.experimental.pallas import tpu_sc as plsc

NC = 2
NS = 16
NW = NC * NS
L = 16

CHUNK = 4096
NBUF = 4
UNROLL = 8
Q = 1024


@functools.partial(jax.jit, static_argnums=(2, 3))
def _time_indexer_sc(time, key_times, n, k):
    ew = n // NW
    nchunks = ew // CHUNK
    ngroups = nchunks // NBUF
    nvec = CHUNK // L

    mesh = plsc.VectorSubcoreMesh(
        core_axis_name="c", subcore_axis_name="s",
        num_cores=NC, num_subcores=NS,
    )

    @functools.partial(
        pl.kernel,
        out_type=(
            jax.ShapeDtypeStruct((n,), jnp.int32),
            jax.ShapeDtypeStruct((n,), jnp.int32),
            jax.ShapeDtypeStruct((n,), jnp.float32),
        ),
        mesh=mesh,
        compiler_params=pltpu.CompilerParams(needs_layout_passes=False),
        scratch_types=[
            pltpu.VMEM((k + L,), jnp.float32),
            pltpu.VMEM((k,), jnp.float32),
            pltpu.VMEM((Q + L,), jnp.int32),
            pltpu.VMEM((Q + L,), jnp.int32),
            [pltpu.VMEM((CHUNK,), jnp.float32) for _ in range(NBUF)],
            [pltpu.VMEM((CHUNK,), jnp.int32) for _ in range(NBUF)],
            [pltpu.VMEM((CHUNK,), jnp.int32) for _ in range(NBUF)],
            [pltpu.VMEM((CHUNK,), jnp.float32) for _ in range(NBUF)],
            [pltpu.SemaphoreType.DMA for _ in range(NBUF)],
            [pltpu.SemaphoreType.DMA for _ in range(NBUF)],
        ],
    )
    def sc_kernel(time_hbm, key_hbm, lo_hbm, up_hbm, fr_hbm,
                  key_v, rtab_v, lut_v, b_v, in_v, lo_v, up_v, fr_v,
                  in_sem, out_sem):
        wid = lax.axis_index("s") * NC + lax.axis_index("c")
        wbase = wid * ew

        pltpu.sync_copy(key_hbm, key_v.at[pl.ds(0, k)])
        maxk = plsc.load_gather(key_v, [jnp.full((L,), k - 1, jnp.int32)])
        key_v[pl.ds(k, L)] = maxk
        mink = jnp.full((L,), jnp.min(key_v[pl.ds(0, L)]), jnp.float32)
        piv1 = plsc.load_gather(key_v, [jnp.full((L,), k // 2 - 1, jnp.int32)])
        piv2a = plsc.load_gather(key_v, [jnp.full((L,), k // 4 - 1, jnp.int32)])
        piv2b = plsc.load_gather(
            key_v, [jnp.full((L,), k // 2 + k // 4 - 1, jnp.int32)])
        for j in range(k // L):
            kk = key_v[pl.ds(j * L, L)]
            kn = key_v[pl.ds(j * L + 1, L)]
            rtab_v[pl.ds(j * L, L)] = 1.0 / (kn - kk)

        def search(t):
            cpr = jnp.where(piv1 < t,
                            jnp.full((L,), k // 2 - 1, jnp.int32),
                            jnp.full((L,), -1, jnp.int32))
            g2 = jnp.where(piv1 < t, piv2b, piv2a)
            cpr = jnp.where(g2 < t, cpr + k // 4, cpr)
            s = k // 8
            while s >= 1:
                probe = cpr + s
                g_ = plsc.load_gather(key_v, [probe])
                cpr = jnp.where(g_ < t, probe, cpr)
                s //= 2
            return cpr + 1

        rngv = maxk - mink
        rng_ok = rngv > jnp.zeros((L,), jnp.float32)
        cw = jnp.where(rng_ok, rngv * (1.0 / Q), jnp.ones((L,), jnp.float32))
        invcw = 1.0 / cw
        delta = cw * (1.0 / 256)
        iota = lax.iota(jnp.int32, L)

        def lut_body(q, _):
            cells = (q * L + iota).astype(jnp.float32)
            gp = mink + cells * cw
            lut_v[pl.ds(q * L, L)] = search(gp)
            b_v[pl.ds(q * L, L)] = search(gp + delta)
            return 0

        lax.fori_loop(0, Q // L, lut_body, 0)
        lut_v[pl.ds(Q, L)] = jnp.full((L,), k, jnp.int32)
        b_v[pl.ds(Q, L)] = jnp.full((L,), k, jnp.int32)

        def chk_body(q, carry):
            amax, bmax = carry
            cells = (q * L + iota).astype(jnp.float32)
            gp = mink + cells * cw
            av = search(gp - delta)
            lv = lut_v[pl.ds(q * L, L)]
            bv1 = b_v[pl.ds(q * L + 1, L)]
            return (jnp.maximum(amax, lv - av), jnp.maximum(bmax, bv1 - lv))

        amax, bmax = lax.fori_loop(
            0, Q // L, chk_body,
            (jnp.zeros((L,), jnp.int32), jnp.zeros((L,), jnp.int32)))
        lut_ok = ((jnp.max(amax) == 0) & (jnp.max(bmax) <= 1)
                  & (jnp.max(rngv) > 0.0))

        def finish(t, idx, tlo, tup, tfr, off):
            upper = jnp.minimum(idx, k - 1)
            lower = jnp.where(t < maxk, jnp.maximum(idx - 1, 0),
                              jnp.full((L,), k - 1, jnp.int32))
            tl = plsc.load_gather(key_v, [lower])
            rt = plsc.load_gather(rtab_v, [lower])
            fr = jnp.where(lower != upper, (t - tl) * rt,
                           jnp.zeros((L,), jnp.float32))
            tlo[pl.ds(off, L)] = lower
            tup[pl.ds(off, L)] = upper
            tfr[pl.ds(off, L)] = fr

        def in_copy(c, b):
            base = wbase + c * CHUNK
            return pltpu.make_async_copy(
                time_hbm.at[pl.ds(base, CHUNK)], in_v[b], in_sem[b])

        def out_copies(c, b):
            base = wbase + c * CHUNK
            return (
                pltpu.make_async_copy(lo_v[b], lo_hbm.at[pl.ds(base, CHUNK)],
                                      out_sem[b]),
                pltpu.make_async_copy(up_v[b], up_hbm.at[pl.ds(base, CHUNK)],
                                      out_sem[b]),
                pltpu.make_async_copy(fr_v[b], fr_hbm.at[pl.ds(base, CHUNK)],
                                      out_sem[b]),
            )

        for b in range(NBUF):
            in_copy(b, b).start()

        def group_body(g, _):
            for b in range(NBUF):
                c = g * NBUF + b
                in_copy(c, b).wait()

                @pl.when(g > 0)
                def _wait_prev_out():
                    for cp in out_copies(c - NBUF, b):
                        cp.wait()

                tin, tlo, tup, tfr = in_v[b], lo_v[b], up_v[b], fr_v[b]

                @pl.when(lut_ok)
                def _fast():
                    @plsc.parallel_loop(0, nvec, unroll=UNROLL)
                    def _vec(i):
                        off = i * L
                        t = tin[pl.ds(off, L)]
                        cellf = jnp.maximum(
                            jnp.minimum((t - mink) * invcw,
                                        jnp.full((L,), Q - 1, jnp.float32)),
                            jnp.zeros((L,), jnp.float32))
                        cell = cellf.astype(jnp.int32)
                        idx0 = plsc.load_gather(lut_v, [cell])
                        gk = plsc.load_gather(key_v, [idx0])
                        idx = jnp.where(gk < t, idx0 + 1, idx0)
                        finish(t, idx, tlo, tup, tfr, off)

                @pl.when(jnp.logical_not(lut_ok))
                def _general():
                    @plsc.parallel_loop(0, nvec, unroll=UNROLL)
                    def _vec(i):
                        off = i * L
                        t = tin[pl.ds(off, L)]
                        finish(t, search(t), tlo, tup, tfr, off)

                for cp in out_copies(c, b):
                    cp.start()

                @pl.when(g < ngroups - 1)
                def _prefetch_next():
                    in_copy(c + NBUF, b).start()
            return 0

        lax.fori_loop(0, ngroups, group_body, 0)

        for b in range(NBUF):
            for cp in out_copies(nchunks - NBUF + b, b):
                cp.wait()

    return sc_kernel(time, key_times)


def kernel(time, key_times):
    n = time.shape[0]
    k = key_times.shape[0]
    return _time_indexer_sc(time, key_times, n, k)


# speedup vs baseline: 1.0671x; 1.0671x over previous
import functools

import jax
import jax.numpy as jnp
from jax import lax
from jax.experimental import pallas as pl
from jax.experimental.pallas import tpu as pltpu
from jax.experimental.pallas import tpu_sc as plsc

NC = 2
NS = 16
NW = NC * NS
L = 16

CHUNK = 8192
NBUF = 2
UNROLL = 8
Q = 1024


@functools.partial(jax.jit, static_argnums=(2, 3))
def _time_indexer_sc(time, key_times, n, k):
    ew = n // NW
    nchunks = ew // CHUNK
    ngroups = nchunks // NBUF
    nvec = CHUNK // L

    mesh = plsc.VectorSubcoreMesh(
        core_axis_name="c", subcore_axis_name="s",
        num_cores=NC, num_subcores=NS,
    )

    @functools.partial(
        pl.kernel,
        out_type=(
            jax.ShapeDtypeStruct((n,), jnp.int32),
            jax.ShapeDtypeStruct((n,), jnp.int32),
            jax.ShapeDtypeStruct((n,), jnp.float32),
        ),
        mesh=mesh,
        compiler_params=pltpu.CompilerParams(needs_layout_passes=False),
        scratch_types=[
            pltpu.VMEM((k + L,), jnp.float32),
            pltpu.VMEM((k,), jnp.float32),
            pltpu.VMEM((Q + L,), jnp.int32),
            pltpu.VMEM((Q + L,), jnp.int32),
            [pltpu.VMEM((CHUNK,), jnp.float32) for _ in range(NBUF)],
            [pltpu.VMEM((CHUNK,), jnp.int32) for _ in range(NBUF)],
            [pltpu.VMEM((CHUNK,), jnp.int32) for _ in range(NBUF)],
            [pltpu.VMEM((CHUNK,), jnp.float32) for _ in range(NBUF)],
            [pltpu.SemaphoreType.DMA for _ in range(NBUF)],
            [pltpu.SemaphoreType.DMA for _ in range(NBUF)],
        ],
    )
    def sc_kernel(time_hbm, key_hbm, lo_hbm, up_hbm, fr_hbm,
                  key_v, rtab_v, lut_v, b_v, in_v, lo_v, up_v, fr_v,
                  in_sem, out_sem):
        wid = lax.axis_index("s") * NC + lax.axis_index("c")
        wbase = wid * ew

        pltpu.sync_copy(key_hbm, key_v.at[pl.ds(0, k)])
        maxk = plsc.load_gather(key_v, [jnp.full((L,), k - 1, jnp.int32)])
        key_v[pl.ds(k, L)] = maxk
        mink = jnp.full((L,), jnp.min(key_v[pl.ds(0, L)]), jnp.float32)
        piv1 = plsc.load_gather(key_v, [jnp.full((L,), k // 2 - 1, jnp.int32)])
        piv2a = plsc.load_gather(key_v, [jnp.full((L,), k // 4 - 1, jnp.int32)])
        piv2b = plsc.load_gather(
            key_v, [jnp.full((L,), k // 2 + k // 4 - 1, jnp.int32)])
        for j in range(k // L):
            kk = key_v[pl.ds(j * L, L)]
            kn = key_v[pl.ds(j * L + 1, L)]
            rtab_v[pl.ds(j * L, L)] = 1.0 / (kn - kk)

        def search(t):
            cpr = jnp.where(piv1 < t,
                            jnp.full((L,), k // 2 - 1, jnp.int32),
                            jnp.full((L,), -1, jnp.int32))
            g2 = jnp.where(piv1 < t, piv2b, piv2a)
            cpr = jnp.where(g2 < t, cpr + k // 4, cpr)
            s = k // 8
            while s >= 1:
                probe = cpr + s
                g_ = plsc.load_gather(key_v, [probe])
                cpr = jnp.where(g_ < t, probe, cpr)
                s //= 2
            return cpr + 1

        rngv = maxk - mink
        rng_ok = rngv > jnp.zeros((L,), jnp.float32)
        cw = jnp.where(rng_ok, rngv * (1.0 / Q), jnp.ones((L,), jnp.float32))
        invcw = 1.0 / cw
        delta = cw * (1.0 / 256)
        iota = lax.iota(jnp.int32, L)

        def lut_body(q, _):
            cells = (q * L + iota).astype(jnp.float32)
            gp = mink + cells * cw
            lut_v[pl.ds(q * L, L)] = search(gp)
            b_v[pl.ds(q * L, L)] = search(gp + delta)
            return 0

        lax.fori_loop(0, Q // L, lut_body, 0)
        lut_v[pl.ds(Q, L)] = jnp.full((L,), k, jnp.int32)
        b_v[pl.ds(Q, L)] = jnp.full((L,), k, jnp.int32)

        def chk_body(q, carry):
            amax, bmax = carry
            cells = (q * L + iota).astype(jnp.float32)
            gp = mink + cells * cw
            av = search(gp - delta)
            lv = lut_v[pl.ds(q * L, L)]
            bv1 = b_v[pl.ds(q * L + 1, L)]
            return (jnp.maximum(amax, lv - av), jnp.maximum(bmax, bv1 - lv))

        amax, bmax = lax.fori_loop(
            0, Q // L, chk_body,
            (jnp.zeros((L,), jnp.int32), jnp.zeros((L,), jnp.int32)))
        lut_ok = ((jnp.max(amax) == 0) & (jnp.max(bmax) <= 1)
                  & (jnp.max(rngv) > 0.0))

        def finish(t, idx, tlo, tup, tfr, off):
            upper = jnp.minimum(idx, k - 1)
            lower = jnp.where(t < maxk, jnp.maximum(idx - 1, 0),
                              jnp.full((L,), k - 1, jnp.int32))
            tl = plsc.load_gather(key_v, [lower])
            rt = plsc.load_gather(rtab_v, [lower])
            fr = jnp.where(lower != upper, (t - tl) * rt,
                           jnp.zeros((L,), jnp.float32))
            tlo[pl.ds(off, L)] = lower
            tup[pl.ds(off, L)] = upper
            tfr[pl.ds(off, L)] = fr

        def in_copy(c, b):
            base = wbase + c * CHUNK
            return pltpu.make_async_copy(
                time_hbm.at[pl.ds(base, CHUNK)], in_v[b], in_sem[b])

        def out_copies(c, b):
            base = wbase + c * CHUNK
            return (
                pltpu.make_async_copy(lo_v[b], lo_hbm.at[pl.ds(base, CHUNK)],
                                      out_sem[b]),
                pltpu.make_async_copy(up_v[b], up_hbm.at[pl.ds(base, CHUNK)],
                                      out_sem[b]),
                pltpu.make_async_copy(fr_v[b], fr_hbm.at[pl.ds(base, CHUNK)],
                                      out_sem[b]),
            )

        for b in range(NBUF):
            in_copy(b, b).start()

        def group_body(g, _):
            for b in range(NBUF):
                c = g * NBUF + b
                in_copy(c, b).wait()

                @pl.when(g > 0)
                def _wait_prev_out():
                    for cp in out_copies(c - NBUF, b):
                        cp.wait()

                tin, tlo, tup, tfr = in_v[b], lo_v[b], up_v[b], fr_v[b]

                @pl.when(lut_ok)
                def _fast():
                    @plsc.parallel_loop(0, nvec, unroll=UNROLL)
                    def _vec(i):
                        off = i * L
                        t = tin[pl.ds(off, L)]
                        cellf = jnp.maximum(
                            jnp.minimum((t - mink) * invcw,
                                        jnp.full((L,), Q - 1, jnp.float32)),
                            jnp.zeros((L,), jnp.float32))
                        cell = cellf.astype(jnp.int32)
                        idx0 = plsc.load_gather(lut_v, [cell])
                        gk = plsc.load_gather(key_v, [idx0])
                        idx = jnp.where(gk < t, idx0 + 1, idx0)
                        finish(t, idx, tlo, tup, tfr, off)

                @pl.when(jnp.logical_not(lut_ok))
                def _general():
                    @plsc.parallel_loop(0, nvec, unroll=UNROLL)
                    def _vec(i):
                        off = i * L
                        t = tin[pl.ds(off, L)]
                        finish(t, search(t), tlo, tup, tfr, off)

                for cp in out_copies(c, b):
                    cp.start()

                @pl.when(g < ngroups - 1)
                def _prefetch_next():
                    in_copy(c + NBUF, b).start()
            return 0

        lax.fori_loop(0, ngroups, group_body, 0)

        for b in range(NBUF):
            for cp in out_copies(nchunks - NBUF + b, b):
                cp.wait()

    return sc_kernel(time, key_times)


def kernel(time, key_times):
    n = time.shape[0]
    k = key_times.shape[0]
    return _time_indexer_sc(time, key_times, n, k)


# speedup vs baseline: 1.0926x; 1.0239x over previous
import functools

import jax
import jax.numpy as jnp
from jax import lax
from jax.experimental import pallas as pl
from jax.experimental.pallas import tpu as pltpu
from jax.experimental.pallas import tpu_sc as plsc

NC = 2
NS = 16
NW = NC * NS
L = 16

CHUNK = 8192
NBUF = 2
UNROLL = 8
Q = 1024


@functools.partial(jax.jit, static_argnums=(2, 3))
def _time_indexer_sc(time, key_times, n, k):
    ew = n // NW
    nchunks = ew // CHUNK
    ngroups = nchunks // NBUF
    nvec = CHUNK // L

    mesh = plsc.VectorSubcoreMesh(
        core_axis_name="c", subcore_axis_name="s",
        num_cores=NC, num_subcores=NS,
    )

    @functools.partial(
        pl.kernel,
        out_type=(
            jax.ShapeDtypeStruct((n,), jnp.int32),
            jax.ShapeDtypeStruct((n,), jnp.int32),
            jax.ShapeDtypeStruct((n,), jnp.float32),
        ),
        mesh=mesh,
        compiler_params=pltpu.CompilerParams(needs_layout_passes=False),
        scratch_types=[
            pltpu.VMEM((k + L,), jnp.float32),
            pltpu.VMEM((k,), jnp.float32),
            pltpu.VMEM((Q + L,), jnp.int32),
            pltpu.VMEM((Q + L,), jnp.int32),
            [pltpu.VMEM((CHUNK,), jnp.float32) for _ in range(NBUF)],
            [pltpu.VMEM((CHUNK,), jnp.int32) for _ in range(NBUF)],
            [pltpu.VMEM((CHUNK,), jnp.int32) for _ in range(NBUF)],
            [pltpu.VMEM((CHUNK,), jnp.float32) for _ in range(NBUF)],
            [pltpu.SemaphoreType.DMA for _ in range(NBUF)],
            [pltpu.SemaphoreType.DMA for _ in range(NBUF)],
        ],
    )
    def sc_kernel(time_hbm, key_hbm, lo_hbm, up_hbm, fr_hbm,
                  key_v, rtab_v, lut_v, b_v, in_v, lo_v, up_v, fr_v,
                  in_sem, out_sem):
        wid = lax.axis_index("s") * NC + lax.axis_index("c")
        wbase = wid * ew

        pltpu.sync_copy(key_hbm, key_v.at[pl.ds(0, k)])
        maxk = plsc.load_gather(key_v, [jnp.full((L,), k - 1, jnp.int32)])
        key_v[pl.ds(k, L)] = maxk
        mink = jnp.full((L,), jnp.min(key_v[pl.ds(0, L)]), jnp.float32)
        piv1 = plsc.load_gather(key_v, [jnp.full((L,), k // 2 - 1, jnp.int32)])
        piv2a = plsc.load_gather(key_v, [jnp.full((L,), k // 4 - 1, jnp.int32)])
        piv2b = plsc.load_gather(
            key_v, [jnp.full((L,), k // 2 + k // 4 - 1, jnp.int32)])
        for j in range(k // L):
            kk = key_v[pl.ds(j * L, L)]
            kn = key_v[pl.ds(j * L + 1, L)]
            rtab_v[pl.ds(j * L, L)] = 1.0 / (kn - kk)
        iota = lax.iota(jnp.int32, L)
        lastr = jnp.where(iota == L - 1, jnp.zeros((L,), jnp.float32),
                          rtab_v[pl.ds(k - L, L)])
        rtab_v[pl.ds(k - L, L)] = lastr

        def search(t):
            cpr = jnp.where(piv1 < t,
                            jnp.full((L,), k // 2 - 1, jnp.int32),
                            jnp.full((L,), -1, jnp.int32))
            g2 = jnp.where(piv1 < t, piv2b, piv2a)
            cpr = jnp.where(g2 < t, cpr + k // 4, cpr)
            s = k // 8
            while s >= 1:
                probe = cpr + s
                g_ = plsc.load_gather(key_v, [probe])
                cpr = jnp.where(g_ < t, probe, cpr)
                s //= 2
            return cpr + 1

        rngv = maxk - mink
        rng_ok = rngv > jnp.zeros((L,), jnp.float32)
        cw = jnp.where(rng_ok, rngv * (1.0 / Q), jnp.ones((L,), jnp.float32))
        invcw = 1.0 / cw
        delta = cw * (1.0 / 256)

        def lut_body(q, _):
            cells = (q * L + iota).astype(jnp.float32)
            gp = mink + cells * cw
            lut_v[pl.ds(q * L, L)] = search(gp)
            b_v[pl.ds(q * L, L)] = search(gp + delta)
            return 0

        lax.fori_loop(0, Q // L, lut_body, 0)
        lut_v[pl.ds(Q, L)] = jnp.full((L,), k, jnp.int32)
        b_v[pl.ds(Q, L)] = jnp.full((L,), k, jnp.int32)

        def chk_body(q, carry):
            amax, bmax = carry
            cells = (q * L + iota).astype(jnp.float32)
            gp = mink + cells * cw
            av = search(gp - delta)
            lv = lut_v[pl.ds(q * L, L)]
            bv1 = b_v[pl.ds(q * L + 1, L)]
            return (jnp.maximum(amax, lv - av), jnp.maximum(bmax, bv1 - lv))

        amax, bmax = lax.fori_loop(
            0, Q // L, chk_body,
            (jnp.zeros((L,), jnp.int32), jnp.zeros((L,), jnp.int32)))
        lut_ok = ((jnp.max(amax) == 0) & (jnp.max(bmax) <= 1)
                  & (jnp.max(rngv) > 0.0))

        def finish(t, idx, tlo, tup, tfr, off):
            upper = jnp.minimum(idx, k - 1)
            lower = jnp.where(t < maxk, jnp.maximum(idx - 1, 0),
                              jnp.full((L,), k - 1, jnp.int32))
            tl = plsc.load_gather(key_v, [lower])
            rt = plsc.load_gather(rtab_v, [lower])
            fr = jnp.maximum((t - tl) * rt, jnp.zeros((L,), jnp.float32))
            tlo[pl.ds(off, L)] = lower
            tup[pl.ds(off, L)] = upper
            tfr[pl.ds(off, L)] = fr

        def in_copy(c, b):
            base = wbase + c * CHUNK
            return pltpu.make_async_copy(
                time_hbm.at[pl.ds(base, CHUNK)], in_v[b], in_sem[b])

        def out_copies(c, b):
            base = wbase + c * CHUNK
            return (
                pltpu.make_async_copy(lo_v[b], lo_hbm.at[pl.ds(base, CHUNK)],
                                      out_sem[b]),
                pltpu.make_async_copy(up_v[b], up_hbm.at[pl.ds(base, CHUNK)],
                                      out_sem[b]),
                pltpu.make_async_copy(fr_v[b], fr_hbm.at[pl.ds(base, CHUNK)],
                                      out_sem[b]),
            )

        for b in range(NBUF):
            in_copy(b, b).start()

        def group_body(g, _):
            for b in range(NBUF):
                c = g * NBUF + b
                in_copy(c, b).wait()

                @pl.when(g > 0)
                def _wait_prev_out():
                    for cp in out_copies(c - NBUF, b):
                        cp.wait()

                tin, tlo, tup, tfr = in_v[b], lo_v[b], up_v[b], fr_v[b]

                @pl.when(lut_ok)
                def _fast():
                    @plsc.parallel_loop(0, nvec, unroll=UNROLL)
                    def _vec(i):
                        off = i * L
                        t = tin[pl.ds(off, L)]
                        cellf = jnp.maximum(
                            jnp.minimum((t - mink) * invcw,
                                        jnp.full((L,), Q - 1, jnp.float32)),
                            jnp.zeros((L,), jnp.float32))
                        cell = cellf.astype(jnp.int32)
                        idx0 = plsc.load_gather(lut_v, [cell])
                        gk = plsc.load_gather(key_v, [idx0])
                        idx = jnp.where(gk < t, idx0 + 1, idx0)
                        finish(t, idx, tlo, tup, tfr, off)

                @pl.when(jnp.logical_not(lut_ok))
                def _general():
                    @plsc.parallel_loop(0, nvec, unroll=UNROLL)
                    def _vec(i):
                        off = i * L
                        t = tin[pl.ds(off, L)]
                        finish(t, search(t), tlo, tup, tfr, off)

                for cp in out_copies(c, b):
                    cp.start()

                @pl.when(g < ngroups - 1)
                def _prefetch_next():
                    in_copy(c + NBUF, b).start()
            return 0

        lax.fori_loop(0, ngroups, group_body, 0)

        for b in range(NBUF):
            for cp in out_copies(nchunks - NBUF + b, b):
                cp.wait()

    return sc_kernel(time, key_times)


def kernel(time, key_times):
    n = time.shape[0]
    k = key_times.shape[0]
    return _time_indexer_sc(time, key_times, n, k)
